# baseline (speedup 1.0000x reference)
import jax
import jax.numpy as jnp
from jax import lax
from jax.experimental import pallas as pl
from jax.experimental.pallas import tpu as pltpu
from jax.experimental.pallas import tpu_sc as plsc

B, S, H = 4096, 200, 128
V, P, T, TLD, TD = 128, 512, 2, 1000, 64
EPS = 1e-12

TLD_PAD = 1024
NW = 32
BPW = B // NW
L = 16
NJ = H // L


def _tables_body(tld_emb_ref, w_ref, b_ref, char_ref, type_ref, pos_ref,
                 tldtab_ref, charc_ref, posc_ref):
    tld = (jnp.dot(tld_emb_ref[...], w_ref[...],
                   preferred_element_type=jnp.float32) + b_ref[...])
    tldtab_ref[...] = tld - jnp.mean(tld, axis=-1, keepdims=True)
    cb = char_ref[...] + type_ref[0:1, :]
    charc_ref[...] = cb - jnp.mean(cb, axis=-1, keepdims=True)
    p = pos_ref[...]
    posc_ref[...] = p - jnp.mean(p, axis=-1, keepdims=True)


def _sc_body(char_hbm, pos_hbm, tldtab_hbm, gamma_hbm, beta_hbm, ids_hbm,
             tldid_hbm, out_hbm,
             char_v, pos_v, gamma_v, beta_v, tldid_v, tldrows_v, ids_v,
             out_v, gsem):
    wid = lax.axis_index("s") * 2 + lax.axis_index("c")
    base = wid * BPW

    pltpu.sync_copy(char_hbm, char_v)
    pltpu.sync_copy(pos_hbm.at[pl.ds(0, S)], pos_v)
    pltpu.sync_copy(gamma_hbm, gamma_v)
    pltpu.sync_copy(beta_hbm, beta_v)
    pltpu.sync_copy(tldid_hbm.at[pl.ds(base, BPW)], tldid_v)
    pltpu.async_copy(tldtab_hbm.at[tldid_v], tldrows_v, gsem).wait()

    def row_body(r, carry):
        pltpu.sync_copy(ids_hbm.at[pl.ds((base + r) * S, S)],
                        ids_v.at[pl.ds(0, S)])
        tld8 = [tldrows_v[r, pl.ds(L * j, L)] for j in range(NJ)]

        @plsc.parallel_loop(0, S, unroll=8)
        def tok_body(t):
            tid = ids_v[pl.ds(t, L)][0]
            e = []
            acc = None
            for j in range(NJ):
                ej = (char_v[tid, pl.ds(L * j, L)]
                      + pos_v[t, pl.ds(L * j, L)] + tld8[j])
                e.append(ej)
                sq = ej * ej
                acc = sq if acc is None else acc + sq
            lanes = lax.iota(jnp.int32, L)
            for sh in (1, 2, 4, 8):
                acc = acc + acc.at[lanes ^ sh].get(mode="promise_in_bounds")
            x = acc * (1.0 / H) + EPS
            xi = lax.bitcast_convert_type(x, jnp.int32)
            yi = jnp.int32(0x5F3759DF) - lax.shift_right_logical(xi, 1)
            y = lax.bitcast_convert_type(yi, jnp.float32)
            y = y * (1.5 - 0.5 * x * y * y)
            y = y * (1.5 - 0.5 * x * y * y)
            y = y * (1.5 - 0.5 * x * y * y)
            for j in range(NJ):
                out_v[t, pl.ds(L * j, L)] = (
                    e[j] * y * gamma_v[pl.ds(L * j, L)]
                    + beta_v[pl.ds(L * j, L)])

        pltpu.sync_copy(out_v, out_hbm.at[base + r])
        return carry

    lax.fori_loop(0, BPW, row_body, 0)


def kernel(input_ids, tld_ids, char_emb, pos_emb, type_emb, tld_emb, W_tld, b_tld, gamma, beta):
    tld_emb_p = jnp.pad(tld_emb, ((0, TLD_PAD - TLD), (0, 0)))
    tld_table, char_c, pos_c = pl.pallas_call(
        _tables_body,
        out_shape=(
            jax.ShapeDtypeStruct((TLD_PAD, H), jnp.float32),
            jax.ShapeDtypeStruct((V, H), jnp.float32),
            jax.ShapeDtypeStruct((P, H), jnp.float32),
        ),
    )(tld_emb_p, W_tld, b_tld.reshape(1, H), char_emb, type_emb, pos_emb)

    ids = input_ids.astype(jnp.int32).reshape(B * S)
    tid = tld_ids.astype(jnp.int32).reshape(B)

    sc = pl.kernel(
        _sc_body,
        out_type=jax.ShapeDtypeStruct((B, S, H), jnp.float32),
        mesh=plsc.VectorSubcoreMesh(core_axis_name="c", subcore_axis_name="s"),
        scratch_types=[
            pltpu.VMEM((V, H), jnp.float32),
            pltpu.VMEM((S, H), jnp.float32),
            pltpu.VMEM((H,), jnp.float32),
            pltpu.VMEM((H,), jnp.float32),
            pltpu.VMEM((BPW,), jnp.int32),
            pltpu.VMEM((BPW, H), jnp.float32),
            pltpu.VMEM((S + L,), jnp.int32),
            pltpu.VMEM((S, H), jnp.float32),
            pltpu.SemaphoreType.DMA,
        ],
    )
    return sc(char_c, pos_c, tld_table, gamma, beta, ids, tid)


# speedup vs baseline: 3.2443x; 3.2443x over previous
import jax
import jax.numpy as jnp
from jax import lax
from jax.experimental import pallas as pl
from jax.experimental.pallas import tpu as pltpu
from jax.experimental.pallas import tpu_sc as plsc

B, S, H = 4096, 200, 128
V, P, T, TLD, TD = 128, 512, 2, 1000, 64
EPS = 1e-12

TLD_PAD = 1024
NW = 32
BPW = B // NW
L = 16
NJ = H // L


def _tables_body(tld_emb_ref, w_ref, b_ref, char_ref, type_ref, pos_ref,
                 tldtab_ref, charc_ref, posc_ref):
    tld = (jnp.dot(tld_emb_ref[...], w_ref[...],
                   preferred_element_type=jnp.float32) + b_ref[...])
    tldtab_ref[...] = tld - jnp.mean(tld, axis=-1, keepdims=True)
    cb = char_ref[...] + type_ref[0:1, :]
    charc_ref[...] = cb - jnp.mean(cb, axis=-1, keepdims=True)
    p = pos_ref[...]
    posc_ref[...] = p - jnp.mean(p, axis=-1, keepdims=True)


def _sc_body(char_hbm, pos_hbm, tldtab_hbm, gamma_hbm, beta_hbm, ids_hbm,
             tldid_hbm, out_hbm,
             char_v, pos_v, gamma_v, beta_v, tldid_v, tldrows_v,
             ids0_v, ids1_v, out0_v, out1_v,
             gsem, isem0, isem1, osem0, osem1):
    wid = lax.axis_index("s") * 2 + lax.axis_index("c")
    base = wid * BPW

    pltpu.sync_copy(char_hbm, char_v)
    pltpu.sync_copy(pos_hbm.at[pl.ds(0, S)], pos_v)
    pltpu.sync_copy(gamma_hbm, gamma_v)
    pltpu.sync_copy(beta_hbm, beta_v)
    pltpu.sync_copy(tldid_hbm.at[pl.ds(base, BPW)], tldid_v)
    pltpu.async_copy(tldtab_hbm.at[tldid_v], tldrows_v, gsem).wait()

    def compute_row(r, ids_v, out_v):
        tld8 = [tldrows_v[r, pl.ds(L * j, L)] for j in range(NJ)]

        @plsc.parallel_loop(0, S, unroll=4)
        def tok_body(t):
            tid = ids_v[pl.ds(t, L)][0]
            e = []
            acc = None
            for j in range(NJ):
                ej = (char_v[tid, pl.ds(L * j, L)]
                      + pos_v[t, pl.ds(L * j, L)] + tld8[j])
                e.append(ej)
                sq = ej * ej
                acc = sq if acc is None else acc + sq
            lanes = lax.iota(jnp.int32, L)
            for sh in (1, 2, 4, 8):
                acc = acc + acc.at[lanes ^ sh].get(mode="promise_in_bounds")
            x = acc * (1.0 / H) + EPS
            xi = lax.bitcast_convert_type(x, jnp.int32)
            yi = jnp.int32(0x5F3759DF) - lax.shift_right_logical(xi, 1)
            y = lax.bitcast_convert_type(yi, jnp.float32)
            y = y * (1.5 - 0.5 * x * y * y)
            y = y * (1.5 - 0.5 * x * y * y)
            y = y * (1.5 - 0.5 * x * y * y)
            for j in range(NJ):
                out_v[t, pl.ds(L * j, L)] = (
                    e[j] * y * gamma_v[pl.ds(L * j, L)]
                    + beta_v[pl.ds(L * j, L)])

    def ids_copy(row, ids_v, sem):
        return pltpu.make_async_copy(
            ids_hbm.at[pl.ds((base + row) * S, S)], ids_v.at[pl.ds(0, S)],
            sem)

    def out_copy(row, out_v, sem):
        return pltpu.make_async_copy(out_v, out_hbm.at[base + row], sem)

    ids_copy(0, ids0_v, isem0).start()
    HB = BPW // 2

    def row_pair(rp, carry):
        r0 = 2 * rp
        ids_copy(r0 + 1, ids1_v, isem1).start()
        ids_copy(r0, ids0_v, isem0).wait()

        @pl.when(rp >= 1)
        def _():
            out_copy(r0, out0_v, osem0).wait()
        compute_row(r0, ids0_v, out0_v)
        out_copy(r0, out0_v, osem0).start()

        @pl.when(rp < HB - 1)
        def _():
            ids_copy(r0 + 2, ids0_v, isem0).start()
        ids_copy(r0 + 1, ids1_v, isem1).wait()

        @pl.when(rp >= 1)
        def _():
            out_copy(r0 + 1, out1_v, osem1).wait()
        compute_row(r0 + 1, ids1_v, out1_v)
        out_copy(r0 + 1, out1_v, osem1).start()
        return carry

    lax.fori_loop(0, HB, row_pair, 0)
    out_copy(BPW - 2, out0_v, osem0).wait()
    out_copy(BPW - 1, out1_v, osem1).wait()


def kernel(input_ids, tld_ids, char_emb, pos_emb, type_emb, tld_emb, W_tld, b_tld, gamma, beta):
    tld_emb_p = jnp.pad(tld_emb, ((0, TLD_PAD - TLD), (0, 0)))
    tld_table, char_c, pos_c = pl.pallas_call(
        _tables_body,
        out_shape=(
            jax.ShapeDtypeStruct((TLD_PAD, H), jnp.float32),
            jax.ShapeDtypeStruct((V, H), jnp.float32),
            jax.ShapeDtypeStruct((P, H), jnp.float32),
        ),
    )(tld_emb_p, W_tld, b_tld.reshape(1, H), char_emb, type_emb, pos_emb)

    ids = input_ids.astype(jnp.int32).reshape(B * S)
    tid = tld_ids.astype(jnp.int32).reshape(B)

    sc = pl.kernel(
        _sc_body,
        out_type=jax.ShapeDtypeStruct((B, S, H), jnp.float32),
        mesh=plsc.VectorSubcoreMesh(core_axis_name="c", subcore_axis_name="s"),
        scratch_types=[
            pltpu.VMEM((V, H), jnp.float32),
            pltpu.VMEM((S, H), jnp.float32),
            pltpu.VMEM((H,), jnp.float32),
            pltpu.VMEM((H,), jnp.float32),
            pltpu.VMEM((BPW,), jnp.int32),
            pltpu.VMEM((BPW, H), jnp.float32),
            pltpu.VMEM((S + L,), jnp.int32),
            pltpu.VMEM((S + L,), jnp.int32),
            pltpu.VMEM((S, H), jnp.float32),
            pltpu.VMEM((S, H), jnp.float32),
            pltpu.SemaphoreType.DMA,
            pltpu.SemaphoreType.DMA,
            pltpu.SemaphoreType.DMA,
            pltpu.SemaphoreType.DMA,
            pltpu.SemaphoreType.DMA,
        ],
    )
    return sc(char_c, pos_c, tld_table, gamma, beta, ids, tid)
